# baseline (speedup 1.0000x reference)
import jax
import jax.numpy as jnp
from jax import lax
from jax.experimental import pallas as pl
from jax.experimental.pallas import tpu as pltpu

EMBED_DIMENSION = 300
EMBED_MAX_NORM = 1.0
VOCAB = 100000
BATCH = 1024

N_TILE = 4096


def _fused_kernel(idx_ref, table_ref, w_ref, b_ref, out_ref, raw_ref, ebf_ref, sem):
    @pl.when(pl.program_id(0) == 0)
    def _():
        def issue(g, carry):
            for u in range(8):
                r = g * 8 + u
                pltpu.make_async_copy(
                    table_ref.at[pl.ds(idx_ref[r], 1), pl.ds(256, 44)],
                    raw_ref.at[pl.ds(r, 1), pl.ds(256, 44)],
                    sem,
                ).start()
            return carry

        lax.fori_loop(0, BATCH // 8, issue, 0)

        pltpu.make_async_copy(
            table_ref.at[pl.ds(0, BATCH), pl.ds(256, 44)],
            raw_ref.at[:, pl.ds(256, 44)],
            sem,
        ).wait()

        e = raw_ref[...]
        nrm = jnp.sqrt(jnp.sum(e * e, axis=1, keepdims=True))
        scale = jnp.minimum(1.0, EMBED_MAX_NORM / jnp.maximum(nrm, 1e-7))
        ebf_ref[...] = (e * scale).astype(jnp.bfloat16)

    e = ebf_ref[...]
    w = w_ref[...].astype(jnp.bfloat16)
    acc = jax.lax.dot_general(
        e, w, (((1,), (1,)), ((), ())), preferred_element_type=jnp.float32
    )
    out_ref[...] = acc + b_ref[0, :][None, :]


@jax.jit
def kernel(inputs, emb_table, W, b):
    n_blocks = pl.cdiv(VOCAB, N_TILE)
    b2 = b.reshape(1, VOCAB)
    return pl.pallas_call(
        _fused_kernel,
        grid_spec=pltpu.PrefetchScalarGridSpec(
            num_scalar_prefetch=1,
            grid=(n_blocks,),
            in_specs=[
                pl.BlockSpec(memory_space=pl.ANY),
                pl.BlockSpec((N_TILE, EMBED_DIMENSION), lambda j, idx: (j, 0)),
                pl.BlockSpec((1, N_TILE), lambda j, idx: (0, j)),
            ],
            out_specs=pl.BlockSpec((BATCH, N_TILE), lambda j, idx: (0, j)),
            scratch_shapes=[
                pltpu.VMEM((BATCH, EMBED_DIMENSION), jnp.float32),
                pltpu.VMEM((BATCH, EMBED_DIMENSION), jnp.bfloat16),
                pltpu.SemaphoreType.DMA,
            ],
        ),
        out_shape=jax.ShapeDtypeStruct((BATCH, VOCAB), jnp.float32),
    )(inputs, emb_table, W, b2)


# speedup vs baseline: 1.0014x; 1.0014x over previous
import jax
import jax.numpy as jnp
from jax import lax
from jax.experimental import pallas as pl
from jax.experimental.pallas import tpu as pltpu

EMBED_DIMENSION = 300
EMBED_MAX_NORM = 1.0
VOCAB = 100000
BATCH = 1024

N_TILE = 4096


def _fused_kernel(idx_ref, table_ref, w_ref, b_ref, out_ref, raw_ref, ebf_ref, sem):
    @pl.when(pl.program_id(0) == 0)
    def _():
        def issue(g, carry):
            for u in range(8):
                r = g * 8 + u
                pltpu.make_async_copy(
                    table_ref.at[pl.ds(idx_ref[r], 1), :],
                    raw_ref.at[pl.ds(r, 1), :],
                    sem.at[u],
                ).start()
            return carry

        lax.fori_loop(0, BATCH // 8, issue, 0)

        for u in range(8):
            pltpu.make_async_copy(
                table_ref.at[pl.ds(0, BATCH // 8), :],
                raw_ref.at[pl.ds(0, BATCH // 8), :],
                sem.at[u],
            ).wait()

        e = raw_ref[...]
        nrm = jnp.sqrt(jnp.sum(e * e, axis=1, keepdims=True))
        scale = jnp.minimum(1.0, EMBED_MAX_NORM / jnp.maximum(nrm, 1e-7))
        ebf_ref[...] = (e * scale).astype(jnp.bfloat16)

    e = ebf_ref[...]
    w = w_ref[...].astype(jnp.bfloat16)
    acc = jax.lax.dot_general(
        e, w, (((1,), (1,)), ((), ())), preferred_element_type=jnp.float32
    )
    out_ref[...] = acc + b_ref[0, :][None, :]


@jax.jit
def kernel(inputs, emb_table, W, b):
    n_blocks = pl.cdiv(VOCAB, N_TILE)
    b2 = b.reshape(1, VOCAB)
    return pl.pallas_call(
        _fused_kernel,
        grid_spec=pltpu.PrefetchScalarGridSpec(
            num_scalar_prefetch=1,
            grid=(n_blocks,),
            in_specs=[
                pl.BlockSpec(memory_space=pl.ANY),
                pl.BlockSpec((N_TILE, EMBED_DIMENSION), lambda j, idx: (j, 0)),
                pl.BlockSpec((1, N_TILE), lambda j, idx: (0, j)),
            ],
            out_specs=pl.BlockSpec((BATCH, N_TILE), lambda j, idx: (0, j)),
            scratch_shapes=[
                pltpu.VMEM((BATCH, EMBED_DIMENSION), jnp.float32),
                pltpu.VMEM((BATCH, EMBED_DIMENSION), jnp.bfloat16),
                pltpu.SemaphoreType.DMA((8,)),
            ],
        ),
        out_shape=jax.ShapeDtypeStruct((BATCH, VOCAB), jnp.float32),
    )(inputs, emb_table, W, b2)
